# baseline (speedup 1.0000x reference)
import jax
import jax.numpy as jnp
from jax.experimental import pallas as pl

_BT = 512
_B = 4096
_BF = jnp.bfloat16


def _mm(a, b, out_dtype=jnp.float32):
    return jax.lax.dot_general(a, b, (((1,), (0,)), ((), ())),
                               preferred_element_type=out_dtype)


def _moe_body(x_ref, t1_ref, t2_ref, wh_ref, b1_ref, b2_ref, bh_ref, out_ref):
    xt = jnp.transpose(x_ref[...]).astype(_BF)

    t1 = t1_ref[...]
    h1p = []
    for g in range(2):
        x3 = jnp.concatenate(
            [xt[32 * i:32 * i + 96, :] for i in range(12 * g, 12 * g + 12)],
            axis=1)
        y = _mm(t1, x3).astype(_BF)
        for k in range(6):
            ya = y[:, (2 * k) * _BT:(2 * k + 1) * _BT]
            yb = y[:, (2 * k + 1) * _BT:(2 * k + 2) * _BT]
            m = jnp.maximum(ya, yb)
            m = m.reshape(12, 2, 32, _BT).max(axis=1)
            m = m.reshape(384, _BT) + b1_ref[...]
            h1p.append(jnp.maximum(m, _BF(0)))
    h1 = jnp.stack(h1p, axis=0)

    hp3 = jnp.concatenate(
        [h1[i2:i2 + 3].reshape(1152, _BT) for i2 in range(10)],
        axis=1)
    z = _mm(t2_ref[...], hp3).astype(_BF)
    flats = []
    for k in range(5):
        za = z[:, (2 * k) * _BT:(2 * k + 1) * _BT]
        zb = z[:, (2 * k + 1) * _BT:(2 * k + 2) * _BT]
        m2 = jnp.maximum(za, zb)
        m2 = m2.reshape(5, 2, 64, _BT).max(axis=1)
        m2 = m2.reshape(320, _BT) + b2_ref[...]
        flats.append(jnp.maximum(m2, _BF(0)))
    flat = jnp.concatenate(flats, axis=0)

    s = _mm(wh_ref[...], flat) + bh_ref[...]
    eo = s[0:50]
    gl = s[50:55]
    gmax = jnp.max(gl, axis=0, keepdims=True)
    ge = jnp.exp(gl - gmax)
    g = ge / jnp.sum(ge, axis=0, keepdims=True)

    gr = [g[e:e + 1] for e in range(5)]
    comb = jnp.zeros((10, _BT), jnp.float32)
    for e in range(5):
        rank = jnp.zeros((1, _BT), jnp.float32)
        for j in range(5):
            if j == e:
                continue
            beats = (gr[j] >= gr[e]) if j < e else (gr[j] > gr[e])
            rank = rank + beats.astype(jnp.float32)
        w_e = jnp.where(rank < 3.0, gr[e], 0.0)
        comb = comb + w_e * eo[10 * e:10 * e + 10]

    cmax = jnp.max(comb, axis=0, keepdims=True)
    ce = jnp.exp(comb - cmax)
    out_ref[...] = ce / jnp.sum(ce, axis=0, keepdims=True)


def kernel(x, W1, b1, W2, b2, Wg, bg, We, be):
    f32 = jnp.float32
    x2 = jnp.pad(x.astype(f32).reshape(_B, 28, 28),
                 ((0, 0), (0, 0), (0, 4))).reshape(_B, 896)

    t1_blocks = []
    for di in range(3):
        p = jnp.broadcast_to(W1[di, :, 0, :], (24, 3, 32))
        p = jnp.pad(p, ((0, 0), (0, 26), (0, 0)))
        q = p.reshape(24 * 29, 32)[:24 * 28].reshape(24, 28, 32)
        q = jnp.pad(q, ((0, 0), (0, 4), (0, 0)))
        t1_blocks.append(q.transpose(0, 2, 1).reshape(768, 32))
    T1 = jnp.concatenate(t1_blocks, axis=1).astype(_BF)

    t2_blocks = []
    for di in range(3):
        p = jnp.broadcast_to(W2[di], (10, 3, 32, 64))
        p = jnp.pad(p, ((0, 0), (0, 10), (0, 0), (0, 0)))
        q = p.reshape(10 * 13, 32, 64)[:10 * 12].reshape(10, 12, 32, 64)
        t2_blocks.append(q.transpose(0, 3, 1, 2).reshape(640, 384))
    T2 = jnp.concatenate(t2_blocks, axis=1).astype(_BF)

    WH = jnp.concatenate([We.transpose(0, 2, 1).reshape(50, 1600), Wg.T],
                         axis=0).astype(_BF)

    B1P = jnp.broadcast_to(jnp.tile(b1, 12)[:, None], (384, _BT)).astype(_BF)
    B2P = jnp.broadcast_to(jnp.tile(b2, 5)[:, None], (320, _BT)).astype(_BF)
    bh = jnp.concatenate([be.reshape(50), bg])
    BHB = jnp.broadcast_to(bh[:, None], (55, _BT))

    outT = pl.pallas_call(
        _moe_body,
        grid=(_B // _BT,),
        in_specs=[
            pl.BlockSpec((_BT, 896), lambda t: (t, 0)),
            pl.BlockSpec((768, 96), lambda t: (0, 0)),
            pl.BlockSpec((640, 1152), lambda t: (0, 0)),
            pl.BlockSpec((55, 1600), lambda t: (0, 0)),
            pl.BlockSpec((384, _BT), lambda t: (0, 0)),
            pl.BlockSpec((320, _BT), lambda t: (0, 0)),
            pl.BlockSpec((55, _BT), lambda t: (0, 0)),
        ],
        out_specs=pl.BlockSpec((10, _BT), lambda t: (0, t)),
        out_shape=jax.ShapeDtypeStruct((10, _B), f32),
    )(x2, T1, T2, WH, B1P, B2P, BHB)
    return outT.T


# speedup vs baseline: 1.0187x; 1.0187x over previous
import jax
import jax.numpy as jnp
from jax.experimental import pallas as pl

_BT = 512
_B = 4096
_BF = jnp.bfloat16


def _mm(a, b, out_dtype=jnp.float32):
    return jax.lax.dot_general(a, b, (((1,), (0,)), ((), ())),
                               preferred_element_type=out_dtype)


def _moe_body(x_ref, t1_ref, t2_ref, wh_ref, b1_ref, b2_ref, bh_ref, out_ref):
    xt = jnp.transpose(x_ref[...]).astype(_BF)

    t1 = t1_ref[...]
    h1p = []
    for g in range(2):
        x3 = jnp.concatenate(
            [xt[32 * i:32 * i + 96, :] for i in range(12 * g, 12 * g + 12)],
            axis=1)
        y = _mm(t1, x3)
        for k in range(6):
            ya = y[:, (2 * k) * _BT:(2 * k + 1) * _BT]
            yb = y[:, (2 * k + 1) * _BT:(2 * k + 2) * _BT]
            m = jnp.maximum(ya, yb)
            m = m.reshape(12, 2, 32, _BT).max(axis=1)
            m = m.reshape(384, _BT) + b1_ref[...]
            h1p.append(jnp.maximum(m, 0.0).astype(_BF))
    h1 = jnp.stack(h1p, axis=0)

    hp3 = jnp.concatenate(
        [h1[i2:i2 + 3].reshape(1152, _BT) for i2 in range(10)],
        axis=1)
    z = _mm(t2_ref[...], hp3)
    flats = []
    for k in range(5):
        za = z[:, (2 * k) * _BT:(2 * k + 1) * _BT]
        zb = z[:, (2 * k + 1) * _BT:(2 * k + 2) * _BT]
        m2 = jnp.maximum(za, zb)
        m2 = m2.reshape(5, 2, 64, _BT).max(axis=1)
        m2 = m2.reshape(320, _BT) + b2_ref[...]
        flats.append(jnp.maximum(m2, 0.0).astype(_BF))
    flat = jnp.concatenate(flats, axis=0)

    s = _mm(wh_ref[...], flat) + bh_ref[...]
    eo = s[0:50]
    gl = s[50:55]
    gmax = jnp.max(gl, axis=0, keepdims=True)
    ge = jnp.exp(gl - gmax)
    g = ge / jnp.sum(ge, axis=0, keepdims=True)

    gr = [g[e:e + 1] for e in range(5)]
    comb = jnp.zeros((10, _BT), jnp.float32)
    for e in range(5):
        rank = jnp.zeros((1, _BT), jnp.float32)
        for j in range(5):
            if j == e:
                continue
            beats = (gr[j] >= gr[e]) if j < e else (gr[j] > gr[e])
            rank = rank + beats.astype(jnp.float32)
        w_e = jnp.where(rank < 3.0, gr[e], 0.0)
        comb = comb + w_e * eo[10 * e:10 * e + 10]

    cmax = jnp.max(comb, axis=0, keepdims=True)
    ce = jnp.exp(comb - cmax)
    out_ref[...] = ce / jnp.sum(ce, axis=0, keepdims=True)


def kernel(x, W1, b1, W2, b2, Wg, bg, We, be):
    f32 = jnp.float32
    x2 = jnp.pad(x.astype(f32).reshape(_B, 28, 28),
                 ((0, 0), (0, 0), (0, 4))).reshape(_B, 896)

    t1_blocks = []
    for di in range(3):
        p = jnp.broadcast_to(W1[di, :, 0, :], (24, 3, 32))
        p = jnp.pad(p, ((0, 0), (0, 26), (0, 0)))
        q = p.reshape(24 * 29, 32)[:24 * 28].reshape(24, 28, 32)
        q = jnp.pad(q, ((0, 0), (0, 4), (0, 0)))
        t1_blocks.append(q.transpose(0, 2, 1).reshape(768, 32))
    T1 = jnp.concatenate(t1_blocks, axis=1).astype(_BF)

    t2_blocks = []
    for di in range(3):
        p = jnp.broadcast_to(W2[di], (10, 3, 32, 64))
        p = jnp.pad(p, ((0, 0), (0, 10), (0, 0), (0, 0)))
        q = p.reshape(10 * 13, 32, 64)[:10 * 12].reshape(10, 12, 32, 64)
        t2_blocks.append(q.transpose(0, 3, 1, 2).reshape(640, 384))
    T2 = jnp.concatenate(t2_blocks, axis=1).astype(_BF)

    WH = jnp.concatenate([We.transpose(0, 2, 1).reshape(50, 1600), Wg.T],
                         axis=0).astype(_BF)

    B1P = jnp.broadcast_to(jnp.tile(b1, 12)[:, None], (384, _BT))
    B2P = jnp.broadcast_to(jnp.tile(b2, 5)[:, None], (320, _BT))
    bh = jnp.concatenate([be.reshape(50), bg])
    BHB = jnp.broadcast_to(bh[:, None], (55, _BT))

    outT = pl.pallas_call(
        _moe_body,
        grid=(_B // _BT,),
        in_specs=[
            pl.BlockSpec((_BT, 896), lambda t: (t, 0)),
            pl.BlockSpec((768, 96), lambda t: (0, 0)),
            pl.BlockSpec((640, 1152), lambda t: (0, 0)),
            pl.BlockSpec((55, 1600), lambda t: (0, 0)),
            pl.BlockSpec((384, _BT), lambda t: (0, 0)),
            pl.BlockSpec((320, _BT), lambda t: (0, 0)),
            pl.BlockSpec((55, _BT), lambda t: (0, 0)),
        ],
        out_specs=pl.BlockSpec((10, _BT), lambda t: (0, t)),
        out_shape=jax.ShapeDtypeStruct((10, _B), f32),
    )(x2, T1, T2, WH, B1P, B2P, BHB)
    return outT.T
